# baseline (speedup 1.0000x reference)
import functools

import jax
import jax.numpy as jnp
from jax import lax
from jax.experimental import pallas as pl
from jax.experimental.pallas import tpu as pltpu
from jax.experimental.pallas import tpu_sc as plsc

M, D, B = 100000, 64, 16384
NC, NS, L = 2, 16, 16
NW = NC * NS
R = M // NW
WB = B // L
TBL = 3136
LIST = 3328
CH = 128
NCH = LIST // CH

_mesh = plsc.VectorSubcoreMesh(core_axis_name="c", subcore_axis_name="s")


@functools.partial(
    pl.kernel,
    out_type=(),
    mesh=_mesh,
    scratch_types=[
        pltpu.VMEM((B,), jnp.int32),
        pltpu.VMEM((TBL,), jnp.int32),
        pltpu.VMEM((LIST,), jnp.int32),
        pltpu.VMEM((LIST,), jnp.int32),
        pltpu.VMEM((NCH, CH), jnp.int32),
        pltpu.VMEM((CH, D), jnp.float32),
        pltpu.VMEM((L,), jnp.int32),
        pltpu.SemaphoreType.DMA,
    ],
    compiler_params=pltpu.CompilerParams(
        needs_layout_passes=False, use_tc_tiling_on_sc=False),
)
def _sc_scatter(value_hbm, index_hbm, mem_ref,
                idx_v, tbl, cl_i, cl_m, m2d, buf, nb, sem):
    wid = lax.axis_index("s") * NC + lax.axis_index("c")
    lo = wid * R
    iot = lax.iota(jnp.int32, L)

    pltpu.sync_copy(index_hbm, idx_v)

    @pl.loop(0, TBL // L)
    def _init(k):
        tbl[pl.ds(k * L, L)] = jnp.full((L,), -1, jnp.int32)

    @pl.loop(0, WB)
    def _scan(k):
        idx = idx_v[pl.ds(k * L, L)]
        inr_any = plsc.all_reduce_population_count(
            (idx >= lo) & (idx < lo + R))[0] > 0

        @pl.when(inr_any)
        def _():
            comb = (idx << 14) | (k * L + iot)
            s = jnp.sort(comb)
            nb[...] = s
            nxt = plsc.load_gather(nb, [jnp.minimum(iot + 1, L - 1)])
            sidx = s >> 14
            winner = (sidx != (nxt >> 14)) | (iot == L - 1)
            inr = (sidx >= lo) & (sidx < lo + R)
            msk = winner & inr
            tgt = jnp.where(msk, sidx - lo, 0)
            plsc.store_scatter(tbl, [tgt], s & (B - 1), mask=msk)

    def _compact(k, off):
        t = tbl[pl.ds(k * L, L)]
        m = lo + k * L + iot
        good = t >= 0
        plsc.store_compressed(cl_i.at[pl.ds(off, L)], t, mask=good)
        plsc.store_compressed(cl_m.at[pl.ds(off, L)], m, mask=good)
        return off + plsc.all_reduce_population_count(good)[0]

    K = lax.fori_loop(0, TBL // L, _compact, jnp.int32(0))

    @pl.when(K > 0)
    def _emit():
        last = jnp.full((L,), K - 1, jnp.int32)
        i_last = plsc.load_gather(cl_i, [last])
        m_last = plsc.load_gather(cl_m, [last])
        base0 = (K // L) * L
        for j in range(9):
            base = base0 + j * L
            g = (base + iot) >= K
            cl_i[pl.ds(base, L)] = jnp.where(g, i_last, cl_i[pl.ds(base, L)])
            cl_m[pl.ds(base, L)] = jnp.where(g, m_last, cl_m[pl.ds(base, L)])

        @pl.loop(0, NCH)
        def _restage(r):
            for q in range(CH // L):
                m2d[r, pl.ds(q * L, L)] = cl_m[pl.ds(r * CH + q * L, L)]

        @pl.loop(0, (K + CH - 1) // CH)
        def _move(c):
            pltpu.async_copy(
                value_hbm.at[cl_i.at[pl.ds(c * CH, CH)]], buf, sem).wait()
            pltpu.async_copy(buf, mem_ref.at[m2d.at[c]], sem).wait()


def kernel(mem, value, index):
    idx = index.astype(jnp.int32)
    mem_ref = jax.new_ref(mem)
    _sc_scatter(value, idx, mem_ref)
    return mem_ref[...]
